# Initial kernel scaffold; baseline (speedup 1.0000x reference)
import jax
import jax.numpy as jnp
from jax.experimental import pallas as pl


def kernel(coords, entropy_spatial):
    raise NotImplementedError("write your pallas kernel here")


# speedup vs baseline: 3.4705x; 3.4705x over previous
import functools

import jax
import jax.numpy as jnp
from jax import lax
from jax.experimental import pallas as pl
from jax.experimental.pallas import tpu as pltpu
from jax.experimental.pallas import tpu_sc as plsc

_N = 16384
_B = 64
_R = 64
_SIDE = 8
_NC = 2
_NS = 16
_NW = _NC * _NS
_CHUNK = _N // _NS
_ROWS_PER_W = _B // _NW


def _body(x_hbm, y_hbm, e_hbm, ids_hbm, w_hbm,
          xv, yv, idv, effv, cacc, cpart, idsv, e0, e1, acc0, acc1,
          cbuf, wv0, wv1, eff_sh, c_sh):
    c = lax.axis_index("c")
    s = lax.axis_index("s")
    w = c * _NS + s
    lane = lax.iota(jnp.int32, 16)
    zeros16 = jnp.zeros((16,), jnp.float32)
    ones16 = jnp.full((16,), 1.0, jnp.float32)

    base = s * _CHUNK
    pltpu.sync_copy(x_hbm.at[pl.ds(base, _CHUNK)], xv)
    pltpu.sync_copy(y_hbm.at[pl.ds(base, _CHUNK)], yv)

    def init1(i, carry):
        cacc[pl.ds(i * 16, 16)] = zeros16
        return carry
    lax.fori_loop(0, _R, init1, None)

    def p1(i, carry):
        o = i * 16
        x = xv[pl.ds(o, 16)]
        y = yv[pl.ds(o, 16)]
        cx = jnp.clip((x * float(_SIDE)).astype(jnp.int32), 0, _SIDE - 1)
        cy = jnp.clip((y * float(_SIDE)).astype(jnp.int32), 0, _SIDE - 1)
        rid = jnp.minimum(cy * _SIDE + cx, _R - 1)
        idv[pl.ds(o, 16)] = rid
        eff = rid * 16 + lane
        effv[pl.ds(o, 16)] = eff
        plsc.addupdate_scatter(cacc, [eff], ones16)
        return carry
    lax.fori_loop(0, _CHUNK // 16, p1, None)

    for g in range(_R // 16):
        bidx = (g * 16 + lane) * 16
        acc = zeros16
        for l in range(16):
            acc = acc + plsc.load_gather(cacc, [bidx + l])
        cpart[pl.ds(g * 16, 16)] = acc

    @pl.when(c == 0)
    def _():
        pltpu.sync_copy(idv, ids_hbm.at[pl.ds(base, _CHUNK)])

    pltpu.sync_copy(effv, eff_sh.at[pl.ds(base, _CHUNK)])
    pltpu.sync_copy(cpart, c_sh.at[pl.ds(s * _R, _R)])
    plsc.subcore_barrier()

    pltpu.sync_copy(eff_sh, idsv)
    pltpu.sync_copy(c_sh, cbuf)
    b0 = w * _ROWS_PER_W
    b1 = b0 + 1
    pltpu.sync_copy(e_hbm.at[b0], e0)
    pltpu.sync_copy(e_hbm.at[b1], e1)

    def init2(i, carry):
        acc0[pl.ds(i * 16, 16)] = zeros16
        acc1[pl.ds(i * 16, 16)] = zeros16
        return carry
    lax.fori_loop(0, _R, init2, None)

    def p2(i, carry):
        o = i * 16
        eff = idsv[pl.ds(o, 16)]
        plsc.addupdate_scatter(acc0, [eff], e0[pl.ds(o, 16)])
        plsc.addupdate_scatter(acc1, [eff], e1[pl.ds(o, 16)])
        return carry
    lax.fori_loop(0, _N // 16, p2, None)

    for g in range(_R // 16):
        o = g * 16
        cn = zeros16
        for t in range(_NS):
            cn = cn + cbuf[pl.ds(t * _R + o, 16)]
        s0 = zeros16
        s1 = zeros16
        bidx = (o + lane) * 16
        for l in range(16):
            s0 = s0 + plsc.load_gather(acc0, [bidx + l])
            s1 = s1 + plsc.load_gather(acc1, [bidx + l])
        safe = jnp.maximum(cn, 1.0)
        nz = cn > 0.0
        wv0[pl.ds(o, 16)] = jnp.where(nz, s0 / safe, 0.0)
        wv1[pl.ds(o, 16)] = jnp.where(nz, s1 / safe, 0.0)

    pltpu.sync_copy(wv0, w_hbm.at[b0])
    pltpu.sync_copy(wv1, w_hbm.at[b1])


_voronoi_sc = functools.partial(
    pl.kernel,
    out_type=[
        jax.ShapeDtypeStruct((_N,), jnp.int32),
        jax.ShapeDtypeStruct((_B, _R), jnp.float32),
    ],
    mesh=plsc.VectorSubcoreMesh(core_axis_name="c", subcore_axis_name="s"),
    compiler_params=pltpu.CompilerParams(needs_layout_passes=False),
    scratch_types=[
        pltpu.VMEM((_CHUNK,), jnp.float32),
        pltpu.VMEM((_CHUNK,), jnp.float32),
        pltpu.VMEM((_CHUNK,), jnp.int32),
        pltpu.VMEM((_CHUNK,), jnp.int32),
        pltpu.VMEM((_R * 16,), jnp.float32),
        pltpu.VMEM((_R,), jnp.float32),
        pltpu.VMEM((_N,), jnp.int32),
        pltpu.VMEM((_N,), jnp.float32),
        pltpu.VMEM((_N,), jnp.float32),
        pltpu.VMEM((_R * 16,), jnp.float32),
        pltpu.VMEM((_R * 16,), jnp.float32),
        pltpu.VMEM((_NS * _R,), jnp.float32),
        pltpu.VMEM((_R,), jnp.float32),
        pltpu.VMEM((_R,), jnp.float32),
        pltpu.VMEM_SHARED((_N,), jnp.int32),
        pltpu.VMEM_SHARED((_NS * _R,), jnp.float32),
    ],
)(_body)


def kernel(coords, entropy_spatial):
    xs = coords[:, 0]
    ys = coords[:, 1]
    ids, weights = _voronoi_sc(xs, ys, entropy_spatial)
    return ids.astype(jnp.int64), weights


# speedup vs baseline: 4.8000x; 1.3831x over previous
import functools

import jax
import jax.numpy as jnp
from jax import lax
from jax.experimental import pallas as pl
from jax.experimental.pallas import tpu as pltpu
from jax.experimental.pallas import tpu_sc as plsc

_N = 16384
_B = 64
_R = 64
_SIDE = 8
_NC = 2
_NS = 16
_NW = _NC * _NS
_CHUNK = _N // _NS
_ROWS_PER_W = _B // _NW


def _body(x_hbm, y_hbm, e_hbm, ids_hbm, w_hbm,
          xv, yv, idv, effv, cacc, cpart, idsv, e0, e1, acc0, acc1,
          cbuf, wv0, wv1, sem, eff_sh, c_sh):
    c = lax.axis_index("c")
    s = lax.axis_index("s")
    w = c * _NS + s
    lane = lax.iota(jnp.int32, 16)
    zeros16 = jnp.zeros((16,), jnp.float32)
    ones16 = jnp.full((16,), 1.0, jnp.float32)

    b0 = w * _ROWS_PER_W
    b1 = b0 + 1
    h_e0 = pltpu.async_copy(e_hbm.at[b0], e0, sem)
    h_e1 = pltpu.async_copy(e_hbm.at[b1], e1, sem)

    base = s * _CHUNK
    h_x = pltpu.async_copy(x_hbm.at[pl.ds(base, _CHUNK)], xv, sem)
    h_y = pltpu.async_copy(y_hbm.at[pl.ds(base, _CHUNK)], yv, sem)

    @plsc.parallel_loop(0, _R, unroll=8)
    def init1(i):
        cacc[pl.ds(i * 16, 16)] = zeros16

    h_x.wait()
    h_y.wait()

    @plsc.parallel_loop(0, _CHUNK // 16, unroll=8)
    def p1(i):
        o = i * 16
        x = xv[pl.ds(o, 16)]
        y = yv[pl.ds(o, 16)]
        cx = jnp.clip((x * float(_SIDE)).astype(jnp.int32), 0, _SIDE - 1)
        cy = jnp.clip((y * float(_SIDE)).astype(jnp.int32), 0, _SIDE - 1)
        rid = jnp.minimum(cy * _SIDE + cx, _R - 1)
        idv[pl.ds(o, 16)] = rid
        eff = rid * 16 + lane
        effv[pl.ds(o, 16)] = eff
        plsc.addupdate_scatter(cacc, [eff], ones16)

    for g in range(_R // 16):
        bidx = (g * 16 + lane) * 16
        acc = zeros16
        for l in range(16):
            acc = acc + plsc.load_gather(cacc, [bidx + l])
        cpart[pl.ds(g * 16, 16)] = acc

    @pl.when(c == 0)
    def _():
        pltpu.sync_copy(idv, ids_hbm.at[pl.ds(base, _CHUNK)])

    pltpu.sync_copy(effv, eff_sh.at[pl.ds(base, _CHUNK)])
    pltpu.sync_copy(cpart, c_sh.at[pl.ds(s * _R, _R)])
    plsc.subcore_barrier()

    h_ids = pltpu.async_copy(eff_sh, idsv, sem)
    h_cb = pltpu.async_copy(c_sh, cbuf, sem)

    @plsc.parallel_loop(0, _R, unroll=8)
    def init2(i):
        acc0[pl.ds(i * 16, 16)] = zeros16
        acc1[pl.ds(i * 16, 16)] = zeros16

    h_ids.wait()
    h_cb.wait()
    h_e0.wait()
    h_e1.wait()

    @plsc.parallel_loop(0, _N // 16, unroll=8)
    def p2(i):
        o = i * 16
        eff = idsv[pl.ds(o, 16)]
        plsc.addupdate_scatter(acc0, [eff], e0[pl.ds(o, 16)])
        plsc.addupdate_scatter(acc1, [eff], e1[pl.ds(o, 16)])

    for g in range(_R // 16):
        o = g * 16
        cn = zeros16
        for t in range(_NS):
            cn = cn + cbuf[pl.ds(t * _R + o, 16)]
        s0 = zeros16
        s1 = zeros16
        bidx = (o + lane) * 16
        for l in range(16):
            s0 = s0 + plsc.load_gather(acc0, [bidx + l])
            s1 = s1 + plsc.load_gather(acc1, [bidx + l])
        safe = jnp.maximum(cn, 1.0)
        nz = cn > 0.0
        wv0[pl.ds(o, 16)] = jnp.where(nz, s0 / safe, 0.0)
        wv1[pl.ds(o, 16)] = jnp.where(nz, s1 / safe, 0.0)

    pltpu.sync_copy(wv0, w_hbm.at[b0])
    pltpu.sync_copy(wv1, w_hbm.at[b1])


_voronoi_sc = functools.partial(
    pl.kernel,
    out_type=[
        jax.ShapeDtypeStruct((_N,), jnp.int32),
        jax.ShapeDtypeStruct((_B, _R), jnp.float32),
    ],
    mesh=plsc.VectorSubcoreMesh(core_axis_name="c", subcore_axis_name="s"),
    compiler_params=pltpu.CompilerParams(needs_layout_passes=False),
    scratch_types=[
        pltpu.VMEM((_CHUNK,), jnp.float32),
        pltpu.VMEM((_CHUNK,), jnp.float32),
        pltpu.VMEM((_CHUNK,), jnp.int32),
        pltpu.VMEM((_CHUNK,), jnp.int32),
        pltpu.VMEM((_R * 16,), jnp.float32),
        pltpu.VMEM((_R,), jnp.float32),
        pltpu.VMEM((_N,), jnp.int32),
        pltpu.VMEM((_N,), jnp.float32),
        pltpu.VMEM((_N,), jnp.float32),
        pltpu.VMEM((_R * 16,), jnp.float32),
        pltpu.VMEM((_R * 16,), jnp.float32),
        pltpu.VMEM((_NS * _R,), jnp.float32),
        pltpu.VMEM((_R,), jnp.float32),
        pltpu.VMEM((_R,), jnp.float32),
        pltpu.SemaphoreType.DMA,
        pltpu.VMEM_SHARED((_N,), jnp.int32),
        pltpu.VMEM_SHARED((_NS * _R,), jnp.float32),
    ],
)(_body)


def kernel(coords, entropy_spatial):
    xs = coords[:, 0]
    ys = coords[:, 1]
    ids, weights = _voronoi_sc(xs, ys, entropy_spatial)
    return ids.astype(jnp.int64), weights
